# baseline (speedup 1.0000x reference)
import functools

import jax
import jax.numpy as jnp
from jax import lax
from jax.experimental import pallas as pl
from jax.experimental.pallas import tpu as pltpu
from jax.experimental.pallas import tpu_sc as plsc

B = 16384
NU = 1000000
D = 32
NC = 2
NS = 16
NW = NC * NS
BPW = B // NW
CHUNK = 128
NCHUNK = BPW // CHUNK
RK = 8192


def _tc_repack_body(guT, muT, giT, miT, out):
    for t, r in enumerate((guT, muT, giT, miT)):
        at = jnp.transpose(r[...].astype(jnp.bfloat16))
        out[:, t * D:(t + 1) * D] = pltpu.bitcast(at, jnp.float32)


def _tc_repack(gu, gi, mu, mi):
    grid = (NU + RK - 1) // RK
    tspec = pl.BlockSpec((D, RK), lambda i: (0, i))
    return pl.pallas_call(
        _tc_repack_body,
        grid=(grid,),
        in_specs=[tspec, tspec, tspec, tspec],
        out_specs=pl.BlockSpec((RK // 2, 4 * D), lambda i: (i, 0)),
        out_shape=jax.ShapeDtypeStruct((NU // 2, 4 * D), jnp.float32),
    )(gu.T, mu.T, gi.T, mi.T)


def _sc_gather(user, item, mega):
    mesh = plsc.VectorSubcoreMesh(core_axis_name="c", subcore_axis_name="s")

    @functools.partial(
        pl.kernel,
        mesh=mesh,
        compiler_params=pltpu.CompilerParams(use_tc_tiling_on_sc=True),
        out_type=(
            jax.ShapeDtypeStruct((B, 4 * D), jnp.float32),
            jax.ShapeDtypeStruct((B, 4 * D), jnp.float32),
        ),
        scratch_types=[
            pltpu.VMEM((NCHUNK, CHUNK), jnp.int32),
            pltpu.VMEM((NCHUNK, CHUNK), jnp.int32),
            pltpu.VMEM((BPW, 4 * D), jnp.float32),
            pltpu.SemaphoreType.DMA,
        ],
    )
    def k(user_ref, item_ref, mega_ref, fu_out, fi_out, idx_u, idx_i, rows_v, sem):
        wid = lax.axis_index("s") * NC + lax.axis_index("c")
        base = wid * BPW
        row0 = wid * NCHUNK

        pltpu.sync_copy(user_ref.at[pl.ds(row0, NCHUNK)], idx_u)
        pltpu.sync_copy(item_ref.at[pl.ds(row0, NCHUNK)], idx_i)

        for idx, out in ((idx_u, fu_out), (idx_i, fi_out)):
            copies = []
            for c in range(NCHUNK):
                copies.append(pltpu.async_copy(
                    mega_ref.at[idx.at[c]],
                    rows_v.at[pl.ds(c * CHUNK, CHUNK)],
                    sem))
            for cp in copies:
                cp.wait()
            pltpu.sync_copy(rows_v, out.at[pl.ds(base, BPW)])

    return k(user, item, mega)


def _tc_mlp_body(fu, fi, pu, pi, w1t, w2t, b1, b2, wo, bo, out):
    def unpack(v, par):
        u = lax.bitcast_convert_type(v[...], jnp.uint32)
        lo = pltpu.unpack_elementwise(
            u, index=0, packed_dtype=jnp.bfloat16, unpacked_dtype=jnp.float32)
        hi = pltpu.unpack_elementwise(
            u, index=1, packed_dtype=jnp.bfloat16, unpacked_dtype=jnp.float32)
        return jnp.where(par[...] > 0, hi, lo)

    ru = unpack(fu, pu)
    ri = unpack(fi, pi)
    gmf = ru[:, 0:D] * ri[:, 2 * D:3 * D]
    h1 = jnp.dot(ru[:, D:2 * D], w1t[...][:D],
                 preferred_element_type=jnp.float32)
    h1 = h1 + jnp.dot(ri[:, 3 * D:4 * D], w1t[...][D:],
                      preferred_element_type=jnp.float32)
    h1 = jnp.maximum(h1 + b1[...], 0.0)
    h2 = jnp.maximum(
        jnp.dot(h1, w2t[...], preferred_element_type=jnp.float32) + b2[...], 0.0)
    o = jnp.dot(gmf, wo[...][:D], preferred_element_type=jnp.float32)
    o = o + jnp.dot(h2, wo[...][D:], preferred_element_type=jnp.float32)
    out[...] = o + bo[0, 0]


def _tc_mlp(fu, fi, pu, pi, w1t, w2t, b1, b2, wo, bo):
    BLK = 2048
    grid = B // BLK
    full = lambda shape: pl.BlockSpec(shape, lambda i: (0, 0))
    return pl.pallas_call(
        _tc_mlp_body,
        grid=(grid,),
        in_specs=[
            pl.BlockSpec((BLK, 4 * D), lambda i: (i, 0)),
            pl.BlockSpec((BLK, 4 * D), lambda i: (i, 0)),
            pl.BlockSpec((BLK, 1), lambda i: (i, 0)),
            pl.BlockSpec((BLK, 1), lambda i: (i, 0)),
            full((2 * D, 64)),
            full((64, D)),
            full((1, 64)),
            full((1, D)),
            full((2 * D, 1)),
            full((1, 1)),
        ],
        out_specs=pl.BlockSpec((BLK, 1), lambda i: (i, 0)),
        out_shape=jax.ShapeDtypeStruct((B, 1), jnp.float32),
    )(fu, fi, pu, pi, w1t, w2t, b1, b2, wo, bo)


def kernel(user, item, gmf_user_w, gmf_item_w, mlp_user_w, mlp_item_w,
           W1, b1, W2, b2, Wo, bo):
    mega = _tc_repack(gmf_user_w, gmf_item_w, mlp_user_w, mlp_item_w)

    ui = user.astype(jnp.int32)
    ii = item.astype(jnp.int32)
    user2d = (ui // 2).reshape(NW * NCHUNK, CHUNK)
    item2d = (ii // 2).reshape(NW * NCHUNK, CHUNK)
    fu, fi = _sc_gather(user2d, item2d, mega)

    pu = (ui % 2).reshape(B, 1)
    pi = (ii % 2).reshape(B, 1)
    w1t = W1.T
    w2t = W2.T
    wo = Wo.T
    out = _tc_mlp(fu, fi, pu, pi, w1t, w2t,
                  b1.reshape(1, -1), b2.reshape(1, -1), wo, bo.reshape(1, 1))
    return out[:, 0]


# speedup vs baseline: 1.0058x; 1.0058x over previous
import functools

import jax
import jax.numpy as jnp
from jax import lax
from jax.experimental import pallas as pl
from jax.experimental.pallas import tpu as pltpu
from jax.experimental.pallas import tpu_sc as plsc

B = 16384
NU = 1000000
D = 32
NC = 2
NS = 16
NW = NC * NS
BPW = B // NW
CHUNK = 128
NCHUNK = BPW // CHUNK
RK = 16384


def _tc_repack_body(guT, muT, giT, miT, out):
    for t, r in enumerate((guT, muT, giT, miT)):
        at = jnp.transpose(r[...].astype(jnp.bfloat16))
        out[:, t * D:(t + 1) * D] = pltpu.bitcast(at, jnp.float32)


def _tc_repack(gu, gi, mu, mi):
    grid = (NU + RK - 1) // RK
    tspec = pl.BlockSpec((D, RK), lambda i: (0, i))
    return pl.pallas_call(
        _tc_repack_body,
        grid=(grid,),
        in_specs=[tspec, tspec, tspec, tspec],
        out_specs=pl.BlockSpec((RK // 2, 4 * D), lambda i: (i, 0)),
        out_shape=jax.ShapeDtypeStruct((NU // 2, 4 * D), jnp.float32),
    )(gu.T, mu.T, gi.T, mi.T)


def _sc_gather(user, item, mega):
    mesh = plsc.VectorSubcoreMesh(core_axis_name="c", subcore_axis_name="s")

    @functools.partial(
        pl.kernel,
        mesh=mesh,
        compiler_params=pltpu.CompilerParams(use_tc_tiling_on_sc=True),
        out_type=(
            jax.ShapeDtypeStruct((B, 4 * D), jnp.float32),
            jax.ShapeDtypeStruct((B, 4 * D), jnp.float32),
        ),
        scratch_types=[
            pltpu.VMEM((NCHUNK, CHUNK), jnp.int32),
            pltpu.VMEM((NCHUNK, CHUNK), jnp.int32),
            pltpu.VMEM((BPW, 4 * D), jnp.float32),
            pltpu.SemaphoreType.DMA,
        ],
    )
    def k(user_ref, item_ref, mega_ref, fu_out, fi_out, idx_u, idx_i, rows_v, sem):
        wid = lax.axis_index("s") * NC + lax.axis_index("c")
        base = wid * BPW
        row0 = wid * NCHUNK

        pltpu.sync_copy(user_ref.at[pl.ds(row0, NCHUNK)], idx_u)
        pltpu.sync_copy(item_ref.at[pl.ds(row0, NCHUNK)], idx_i)

        for idx, out in ((idx_u, fu_out), (idx_i, fi_out)):
            copies = []
            for c in range(NCHUNK):
                copies.append(pltpu.async_copy(
                    mega_ref.at[idx.at[c]],
                    rows_v.at[pl.ds(c * CHUNK, CHUNK)],
                    sem))
            for cp in copies:
                cp.wait()
            pltpu.sync_copy(rows_v, out.at[pl.ds(base, BPW)])

    return k(user, item, mega)


def _tc_mlp_body(fu, fi, pu, pi, w1t, w2t, b1, b2, wo, bo, out):
    def unpack(v, par):
        u = lax.bitcast_convert_type(v[...], jnp.uint32)
        lo = pltpu.unpack_elementwise(
            u, index=0, packed_dtype=jnp.bfloat16, unpacked_dtype=jnp.float32)
        hi = pltpu.unpack_elementwise(
            u, index=1, packed_dtype=jnp.bfloat16, unpacked_dtype=jnp.float32)
        return jnp.where(par[...] > 0, hi, lo)

    ru = unpack(fu, pu)
    ri = unpack(fi, pi)
    gmf = ru[:, 0:D] * ri[:, 2 * D:3 * D]
    h1 = jnp.dot(ru[:, D:2 * D], w1t[...][:D],
                 preferred_element_type=jnp.float32)
    h1 = h1 + jnp.dot(ri[:, 3 * D:4 * D], w1t[...][D:],
                      preferred_element_type=jnp.float32)
    h1 = jnp.maximum(h1 + b1[...], 0.0)
    h2 = jnp.maximum(
        jnp.dot(h1, w2t[...], preferred_element_type=jnp.float32) + b2[...], 0.0)
    o = jnp.dot(gmf, wo[...][:D], preferred_element_type=jnp.float32)
    o = o + jnp.dot(h2, wo[...][D:], preferred_element_type=jnp.float32)
    out[...] = o + bo[0, 0]


def _tc_mlp(fu, fi, pu, pi, w1t, w2t, b1, b2, wo, bo):
    BLK = 2048
    grid = B // BLK
    full = lambda shape: pl.BlockSpec(shape, lambda i: (0, 0))
    return pl.pallas_call(
        _tc_mlp_body,
        grid=(grid,),
        in_specs=[
            pl.BlockSpec((BLK, 4 * D), lambda i: (i, 0)),
            pl.BlockSpec((BLK, 4 * D), lambda i: (i, 0)),
            pl.BlockSpec((BLK, 1), lambda i: (i, 0)),
            pl.BlockSpec((BLK, 1), lambda i: (i, 0)),
            full((2 * D, 64)),
            full((64, D)),
            full((1, 64)),
            full((1, D)),
            full((2 * D, 1)),
            full((1, 1)),
        ],
        out_specs=pl.BlockSpec((BLK, 1), lambda i: (i, 0)),
        out_shape=jax.ShapeDtypeStruct((B, 1), jnp.float32),
    )(fu, fi, pu, pi, w1t, w2t, b1, b2, wo, bo)


def kernel(user, item, gmf_user_w, gmf_item_w, mlp_user_w, mlp_item_w,
           W1, b1, W2, b2, Wo, bo):
    mega = _tc_repack(gmf_user_w, gmf_item_w, mlp_user_w, mlp_item_w)

    ui = user.astype(jnp.int32)
    ii = item.astype(jnp.int32)
    user2d = (ui // 2).reshape(NW * NCHUNK, CHUNK)
    item2d = (ii // 2).reshape(NW * NCHUNK, CHUNK)
    fu, fi = _sc_gather(user2d, item2d, mega)

    pu = (ui % 2).reshape(B, 1)
    pi = (ii % 2).reshape(B, 1)
    w1t = W1.T
    w2t = W2.T
    wo = Wo.T
    out = _tc_mlp(fu, fi, pu, pi, w1t, w2t,
                  b1.reshape(1, -1), b2.reshape(1, -1), wo, bo.reshape(1, 1))
    return out[:, 0]
